# baseline (speedup 1.0000x reference)
import functools

import jax
import jax.numpy as jnp
from jax import lax
from jax.experimental import pallas as pl
from jax.experimental.pallas import tpu as pltpu
from jax.experimental.pallas import tpu_sc as plsc

B, S, F = 32, 2048, 512
SEG = 256
LANES = 16
WIN = SEG + 8
CHG = 88


def _gather_sc(x, indices, starts):
    mesh = plsc.VectorSubcoreMesh(core_axis_name="c", subcore_axis_name="s")
    info = plsc.get_sparse_core_info()
    nc = info.num_cores

    @functools.partial(
        pl.kernel,
        mesh=mesh,
        out_type=jax.ShapeDtypeStruct((B, WIN, F), jnp.float32),
        scratch_types=(
            [pltpu.VMEM((B + 16,), jnp.int32)] * 2
            + [pltpu.VMEM((CHG, F), jnp.float32)] * 2
            + [pltpu.VMEM((8, F), jnp.float32)] * 2
            + [pltpu.SemaphoreType.DMA] * 8
        ),
    )
    def k(x_hbm, idx_hbm, st_hbm, seg_hbm, idx_v, st_v,
          buf0, buf1, eb0, eb2, g0, g1, g2, e0, e2, s0, s1, s2):
        wid = lax.axis_index("s") * nc + lax.axis_index("c")
        pltpu.sync_copy(idx_hbm, idx_v.at[pl.ds(0, B)])
        pltpu.sync_copy(st_hbm, st_v.at[pl.ds(0, B)])
        p = idx_v[pl.ds(wid, LANES)][0]
        st = st_v[pl.ds(wid, LANES)][0]
        m = lax.rem(st, 8)
        a0 = pl.multiple_of(st - m, 8)

        hg0 = pltpu.async_copy(x_hbm.at[p, pl.ds(a0, CHG)], buf0, g0)
        hg1 = pltpu.async_copy(x_hbm.at[p, pl.ds(a0 + CHG, CHG)], buf1, g1)
        he0 = pltpu.async_copy(x_hbm.at[wid, pl.ds(a0, 8)], eb0, e0)
        he2 = pltpu.async_copy(x_hbm.at[wid, pl.ds(a0 + SEG, 8)], eb2, e2)

        hg0.wait()
        he0.wait()
        for r in range(8):
            @pl.when(r < m)
            def _lead():
                for c in range(F // LANES):
                    sl = pl.ds(c * LANES, LANES)
                    buf0[r, sl] = eb0[r, sl]
        hs0 = pltpu.async_copy(buf0, seg_hbm.at[wid, pl.ds(0, CHG)], s0)

        hg1.wait()
        hs1 = pltpu.async_copy(buf1, seg_hbm.at[wid, pl.ds(CHG, CHG)], s1)

        hs0.wait()
        hg2 = pltpu.async_copy(x_hbm.at[p, pl.ds(a0 + 2 * CHG, CHG)], buf0, g2)
        hg2.wait()
        he2.wait()
        for r in range(8):
            @pl.when(r >= m)
            def _trail():
                for c in range(F // LANES):
                    sl = pl.ds(c * LANES, LANES)
                    buf0[80 + r, sl] = eb2[r, sl]
        hs2 = pltpu.async_copy(buf0, seg_hbm.at[wid, pl.ds(2 * CHG, CHG)], s2)

        hs1.wait()
        hs2.wait()

    return k(x, indices, starts)


NB = 8
AHEAD = 4
NBP = 4


def _bulk_tc(x):
    def body(x_ref, out_ref, *scr):
        bufs = list(scr[:NB])
        in_sem, out_sem = scr[NB:]
        in_h = [None] * B
        out_h = [None] * B
        waited = [False] * B
        for t in range(B + AHEAD):
            if t < B:
                b = t % NB
                if t >= NB:
                    out_h[t - NB].wait()
                    waited[t - NB] = True
                in_h[t] = pltpu.make_async_copy(x_ref.at[t], bufs[b],
                                                in_sem.at[b])
                in_h[t].start()
            u = t - AHEAD
            if 0 <= u < B:
                b = u % NB
                in_h[u].wait()
                out_h[u] = pltpu.make_async_copy(bufs[b], out_ref.at[u],
                                                 out_sem.at[b])
                out_h[u].start()
        for i in range(B):
            if not waited[i]:
                out_h[i].wait()

    return pl.pallas_call(
        body,
        out_shape=jax.ShapeDtypeStruct((B, S, F), jnp.float32),
        in_specs=[pl.BlockSpec(memory_space=pl.ANY)],
        out_specs=pl.BlockSpec(memory_space=pl.ANY),
        scratch_shapes=(
            [pltpu.VMEM((S, F), jnp.float32)] * NB
            + [pltpu.SemaphoreType.DMA((NB,))] * 2
        ),
    )(x)


def _patch_tc(y, seg, starts):
    def body(y_ref, st_ref, seg_ref, out_ref, *scr):
        bufs = list(scr[:NBP])
        in_sem, out_sem = scr[NBP:]
        in_h = [None] * B
        out_h = [None] * B
        waited = [False] * B
        for t in range(B + 1):
            if t < B:
                b = t % NBP
                if t >= NBP:
                    out_h[t - NBP].wait()
                    waited[t - NBP] = True
                in_h[t] = pltpu.make_async_copy(seg_ref.at[t], bufs[b],
                                                in_sem.at[b])
                in_h[t].start()
            u = t - 1
            if 0 <= u < B:
                b = u % NBP
                in_h[u].wait()
                st = st_ref[u]
                a0 = pl.multiple_of(st - lax.rem(st, 8), 8)
                out_h[u] = pltpu.make_async_copy(
                    bufs[b], out_ref.at[u, pl.ds(a0, WIN)], out_sem.at[b])
                out_h[u].start()
        for i in range(B):
            if not waited[i]:
                out_h[i].wait()

    return pl.pallas_call(
        body,
        out_shape=jax.ShapeDtypeStruct((B, S, F), jnp.float32),
        in_specs=[
            pl.BlockSpec(memory_space=pl.ANY),
            pl.BlockSpec(memory_space=pltpu.SMEM),
            pl.BlockSpec(memory_space=pl.ANY),
        ],
        out_specs=pl.BlockSpec(memory_space=pl.ANY),
        input_output_aliases={0: 0},
        scratch_shapes=(
            [pltpu.VMEM((WIN, F), jnp.float32)] * NBP
            + [pltpu.SemaphoreType.DMA((NBP,))] * 2
        ),
    )(y, starts, seg)


def kernel(x, indices, starts):
    seg = _gather_sc(x, indices, starts)
    y = _bulk_tc(x)
    return _patch_tc(y, seg, starts)


# speedup vs baseline: 1.1058x; 1.1058x over previous
import functools

import jax
import jax.numpy as jnp
from jax import lax
from jax.experimental import pallas as pl
from jax.experimental.pallas import tpu as pltpu
from jax.experimental.pallas import tpu_sc as plsc

B, S, F = 32, 2048, 512
SEG = 256
LANES = 16
CH = 16
NCH = S // CH
NBUF = 12


def _cutmix_sc(x, indices, starts):
    mesh = plsc.VectorSubcoreMesh(core_axis_name="c", subcore_axis_name="s")
    info = plsc.get_sparse_core_info()
    nc = info.num_cores

    @functools.partial(
        pl.kernel,
        mesh=mesh,
        out_type=jax.ShapeDtypeStruct((B, S, F), jnp.float32),
        scratch_types=(
            [pltpu.VMEM((B + 16,), jnp.int32)] * 2
            + [pltpu.VMEM((CH, F), jnp.float32)] * NBUF
            + [pltpu.VMEM((8, F), jnp.float32)] * 2
            + [pltpu.SemaphoreType.DMA] * (2 * NBUF)
        ),
    )
    def k(x_hbm, idx_hbm, st_hbm, out_hbm, *scr):
        idx_v, st_v = scr[0], scr[1]
        bufs = scr[2:2 + NBUF]
        buf_i, buf_p = scr[2 + NBUF], scr[3 + NBUF]
        sin = scr[4 + NBUF:4 + 2 * NBUF]
        sout = scr[4 + 2 * NBUF:4 + 3 * NBUF]

        wid = lax.axis_index("s") * nc + lax.axis_index("c")
        pltpu.sync_copy(idx_hbm, idx_v.at[pl.ds(0, B)])
        pltpu.sync_copy(st_hbm, st_v.at[pl.ds(0, B)])
        p = idx_v[pl.ds(wid, LANES)][0]
        st = st_v[pl.ds(wid, LANES)][0]
        m = lax.rem(st, 8)
        q = lax.rem(st, CH)
        g = (q - m) // 8

        def start_in(c):
            b = c % NBUF
            c0 = c * CH
            inside = jnp.logical_and(st <= c0, c0 + CH <= st + SEG)
            src = lax.select(inside, p, wid)
            return pltpu.async_copy(
                x_hbm.at[src, pl.ds(c0, CH)], bufs[b], sin[b])

        def start_out(c):
            b = c % NBUF
            return pltpu.async_copy(
                bufs[b], out_hbm.at[wid, pl.ds(c * CH, CH)], sout[b])

        in_h = [None] * NBUF
        out_h = [None] * NBUF
        for c in range(NBUF - 1):
            in_h[c] = start_in(c)
        for c in range(NCH):
            b = c % NBUF
            if c + NBUF - 1 < NCH:
                b2 = (c + NBUF - 1) % NBUF
                if c >= 1:
                    out_h[b2].wait()
                in_h[b2] = start_in(c + NBUF - 1)
            in_h[b].wait()
            out_h[b] = start_out(c)
        for b in range(min(NBUF, NCH)):
            out_h[b].wait()

        def seg_copy(off, n):
            pltpu.sync_copy(
                x_hbm.at[p, pl.ds(off, n)], bufs[0].at[pl.ds(0, n)])
            pltpu.sync_copy(
                bufs[0].at[pl.ds(0, n)], out_hbm.at[wid, pl.ds(off, n)])

        def copy_8blocks(off, nblocks):
            for j in range(1, CH // 8):
                @pl.when(nblocks == j)
                def _arm():
                    seg_copy(pl.multiple_of(off, 8), 8 * j)

        def merge_edge(base, from_p):
            pltpu.sync_copy(x_hbm.at[wid, pl.ds(base, 8)], buf_i)
            pltpu.sync_copy(x_hbm.at[p, pl.ds(base, 8)], buf_p)
            for r in range(8):
                @pl.when(from_p(r))
                def _row():
                    for c in range(F // LANES):
                        sl = pl.ds(c * LANES, LANES)
                        buf_i[r, sl] = buf_p[r, sl]
            pltpu.sync_copy(buf_i, out_hbm.at[wid, pl.ds(base, 8)])

        @pl.when(jnp.logical_and(q != 0, m == 0))
        def _aligned8():
            copy_8blocks(st, (CH - q) // 8)
            copy_8blocks(st + SEG - q, g)

        @pl.when(m != 0)
        def _unaligned():
            a0 = pl.multiple_of(st - m, 8)
            b0 = pl.multiple_of(st + SEG - m, 8)
            copy_8blocks(a0 + 8, (CH - 8) // 8 - g)
            copy_8blocks(st + SEG - q, g)
            merge_edge(a0, lambda r: r >= m)
            merge_edge(b0, lambda r: r < m)

    return k(x, indices, starts)


def kernel(x, indices, starts):
    return _cutmix_sc(x, indices, starts)


# speedup vs baseline: 1.1080x; 1.0019x over previous
import functools

import jax
import jax.numpy as jnp
from jax import lax
from jax.experimental import pallas as pl
from jax.experimental.pallas import tpu as pltpu
from jax.experimental.pallas import tpu_sc as plsc

B, S, F = 32, 2048, 512
SEG = 256
LANES = 16
CH = 16
NCH = S // CH
NBUF = 14


def _cutmix_sc(x, indices, starts):
    mesh = plsc.VectorSubcoreMesh(core_axis_name="c", subcore_axis_name="s")
    info = plsc.get_sparse_core_info()
    nc = info.num_cores

    @functools.partial(
        pl.kernel,
        mesh=mesh,
        out_type=jax.ShapeDtypeStruct((B, S, F), jnp.float32),
        scratch_types=(
            [pltpu.VMEM((B + 16,), jnp.int32)] * 2
            + [pltpu.VMEM((CH, F), jnp.float32)] * NBUF
            + [pltpu.VMEM((8, F), jnp.float32)] * 2
            + [pltpu.SemaphoreType.DMA] * (2 * NBUF)
        ),
    )
    def k(x_hbm, idx_hbm, st_hbm, out_hbm, *scr):
        idx_v, st_v = scr[0], scr[1]
        bufs = scr[2:2 + NBUF]
        buf_i, buf_p = scr[2 + NBUF], scr[3 + NBUF]
        sin = scr[4 + NBUF:4 + 2 * NBUF]
        sout = scr[4 + 2 * NBUF:4 + 3 * NBUF]

        wid = lax.axis_index("s") * nc + lax.axis_index("c")
        pltpu.sync_copy(idx_hbm, idx_v.at[pl.ds(0, B)])
        pltpu.sync_copy(st_hbm, st_v.at[pl.ds(0, B)])
        p = idx_v[pl.ds(wid, LANES)][0]
        st = st_v[pl.ds(wid, LANES)][0]
        m = lax.rem(st, 8)
        q = lax.rem(st, CH)
        g = (q - m) // 8

        def start_in(c):
            b = c % NBUF
            c0 = c * CH
            inside = jnp.logical_and(st <= c0, c0 + CH <= st + SEG)
            src = lax.select(inside, p, wid)
            return pltpu.async_copy(
                x_hbm.at[src, pl.ds(c0, CH)], bufs[b], sin[b])

        def start_out(c):
            b = c % NBUF
            return pltpu.async_copy(
                bufs[b], out_hbm.at[wid, pl.ds(c * CH, CH)], sout[b])

        in_h = [None] * NBUF
        out_h = [None] * NBUF
        for c in range(NBUF - 1):
            in_h[c] = start_in(c)
        for c in range(NCH):
            b = c % NBUF
            if c + NBUF - 1 < NCH:
                b2 = (c + NBUF - 1) % NBUF
                if c >= 1:
                    out_h[b2].wait()
                in_h[b2] = start_in(c + NBUF - 1)
            in_h[b].wait()
            out_h[b] = start_out(c)
        for b in range(min(NBUF, NCH)):
            out_h[b].wait()

        def seg_copy(off, n):
            pltpu.sync_copy(
                x_hbm.at[p, pl.ds(off, n)], bufs[0].at[pl.ds(0, n)])
            pltpu.sync_copy(
                bufs[0].at[pl.ds(0, n)], out_hbm.at[wid, pl.ds(off, n)])

        def copy_8blocks(off, nblocks):
            for j in range(1, CH // 8):
                @pl.when(nblocks == j)
                def _arm():
                    seg_copy(pl.multiple_of(off, 8), 8 * j)

        def merge_edge(base, from_p):
            pltpu.sync_copy(x_hbm.at[wid, pl.ds(base, 8)], buf_i)
            pltpu.sync_copy(x_hbm.at[p, pl.ds(base, 8)], buf_p)
            for r in range(8):
                @pl.when(from_p(r))
                def _row():
                    for c in range(F // LANES):
                        sl = pl.ds(c * LANES, LANES)
                        buf_i[r, sl] = buf_p[r, sl]
            pltpu.sync_copy(buf_i, out_hbm.at[wid, pl.ds(base, 8)])

        @pl.when(jnp.logical_and(q != 0, m == 0))
        def _aligned8():
            copy_8blocks(st, (CH - q) // 8)
            copy_8blocks(st + SEG - q, g)

        @pl.when(m != 0)
        def _unaligned():
            a0 = pl.multiple_of(st - m, 8)
            b0 = pl.multiple_of(st + SEG - m, 8)
            copy_8blocks(a0 + 8, (CH - 8) // 8 - g)
            copy_8blocks(st + SEG - q, g)
            merge_edge(a0, lambda r: r >= m)
            merge_edge(b0, lambda r: r < m)

    return k(x, indices, starts)


def kernel(x, indices, starts):
    return _cutmix_sc(x, indices, starts)
